# baseline (speedup 1.0000x reference)
import functools

import jax
import jax.numpy as jnp
from jax import lax
from jax.experimental import pallas as pl
from jax.experimental.pallas import tpu as pltpu
from jax.experimental.pallas import tpu_sc as plsc

D_MODEL = 128
BATCH = 4096
NUM_DIM = 512

_NC = 2
_NS = 16
_NW = _NC * _NS
_BPW = BATCH // _NW


def _assemble_sc(i0, i1, i2, i3, num, t0, t1, t2, t3):
    mesh = plsc.VectorSubcoreMesh(core_axis_name="c", subcore_axis_name="s")

    @functools.partial(
        pl.kernel,
        mesh=mesh,
        out_type=jax.ShapeDtypeStruct((6, BATCH, D_MODEL), jnp.float32),
        scratch_types=[
            pltpu.VMEM((4, _BPW), jnp.int32),
            pltpu.VMEM((4, _BPW, D_MODEL), jnp.float32),
            pltpu.SemaphoreType.DMA((4,)),
            pltpu.SemaphoreType.DMA((4,)),
        ],
    )
    def body(ih0, ih1, ih2, ih3, num_hbm, tb0, tb1, tb2, tb3, out_hbm,
             idx_v, rows_v, sem_g, sem_w):
        wid = lax.axis_index("s") * _NC + lax.axis_index("c")
        base = wid * _BPW
        for i, ih in enumerate((ih0, ih1, ih2, ih3)):
            pltpu.sync_copy(ih.at[pl.ds(base, _BPW)], idx_v.at[i])
        gathers = []
        for i, tbl in enumerate((tb0, tb1, tb2, tb3)):
            gathers.append(
                pltpu.async_copy(tbl.at[idx_v.at[i]], rows_v.at[i], sem_g.at[i]))
        for k in range(2):
            pltpu.sync_copy(num_hbm.at[k, pl.ds(base, _BPW)],
                            out_hbm.at[4 + k, pl.ds(base, _BPW)])
        writes = []
        for i, g in enumerate(gathers):
            g.wait()
            writes.append(
                pltpu.async_copy(rows_v.at[i],
                                 out_hbm.at[i, pl.ds(base, _BPW)], sem_w.at[i]))
        for w in writes:
            w.wait()

    return body(i0, i1, i2, i3, num, t0, t1, t2, t3)


def _num_body(x0_ref, x1_ref, g0_ref, be0_ref, w0_ref, b0_ref,
              g1_ref, be1_ref, w1_ref, b1_ref, out_ref):
    j = pl.program_id(0)

    def compute(x, g, be, w, b):
        mean = jnp.mean(x, axis=0, keepdims=True)
        xc = x - mean
        var = jnp.mean(xc * xc, axis=0, keepdims=True)
        h = xc * (g * lax.rsqrt(var + 1e-5)) + be
        out_ref[0] = jnp.dot(h, w, preferred_element_type=jnp.float32) + b

    @pl.when(j == 0)
    def _():
        compute(x0_ref[...], g0_ref[...], be0_ref[...], w0_ref[...], b0_ref[...])

    @pl.when(j == 1)
    def _():
        compute(x1_ref[...], g1_ref[...], be1_ref[...], w1_ref[...], b1_ref[...])


def kernel(x_cat0, x_cat1, x_cat2, x_cat3, x_num0, x_num1,
           table0, table1, table2, table3,
           gamma0, beta0, W0, b0, gamma1, beta1, W1, b1):
    full = pl.BlockSpec(memory_space=pltpu.MemorySpace.VMEM)
    num = pl.pallas_call(
        _num_body,
        grid=(2,),
        in_specs=[full] * 10,
        out_specs=pl.BlockSpec((1, BATCH, D_MODEL), lambda j: (j, 0, 0)),
        out_shape=jax.ShapeDtypeStruct((2, BATCH, D_MODEL), jnp.float32),
    )(x_num0, x_num1,
      gamma0.reshape(1, NUM_DIM), beta0.reshape(1, NUM_DIM), W0,
      b0.reshape(1, D_MODEL),
      gamma1.reshape(1, NUM_DIM), beta1.reshape(1, NUM_DIM), W1,
      b1.reshape(1, D_MODEL))

    return _assemble_sc(x_cat0.astype(jnp.int32), x_cat1.astype(jnp.int32),
                        x_cat2.astype(jnp.int32), x_cat3.astype(jnp.int32),
                        num, table0, table1, table2, table3)


# speedup vs baseline: 3.9796x; 3.9796x over previous
import functools

import jax
import jax.numpy as jnp
from jax import lax
from jax.experimental import pallas as pl
from jax.experimental.pallas import tpu as pltpu
from jax.experimental.pallas import tpu_sc as plsc

D_MODEL = 128
BATCH = 4096
NUM_DIM = 512

_NC = 2
_NS = 16
_NW = _NC * _NS
_BPW = BATCH // _NW


def _assemble_sc(i0, i1, i2, i3, num, t0, t1, t2, t3):
    mesh = plsc.VectorSubcoreMesh(core_axis_name="c", subcore_axis_name="s")

    @functools.partial(
        pl.kernel,
        mesh=mesh,
        out_type=jax.ShapeDtypeStruct((6, BATCH, D_MODEL), jnp.float32),
        scratch_types=[
            pltpu.VMEM((4, _BPW), jnp.int32),
            pltpu.VMEM((4, _BPW, D_MODEL), jnp.float32),
            pltpu.VMEM((2, _BPW, D_MODEL), jnp.float32),
            pltpu.SemaphoreType.DMA((4,)),
            pltpu.SemaphoreType.DMA((2,)),
            pltpu.SemaphoreType.DMA((6,)),
        ],
    )
    def body(ih0, ih1, ih2, ih3, num_hbm, tb0, tb1, tb2, tb3, out_hbm,
             idx_v, rows_v, num_v, sem_g, sem_p, sem_w):
        wid = lax.axis_index("s") * _NC + lax.axis_index("c")
        base = wid * _BPW
        for i, ih in enumerate((ih0, ih1, ih2, ih3)):
            pltpu.sync_copy(ih.at[pl.ds(base, _BPW)], idx_v.at[i])
        gathers = []
        for i, tbl in enumerate((tb0, tb1, tb2, tb3)):
            gathers.append(
                pltpu.async_copy(tbl.at[idx_v.at[i]], rows_v.at[i], sem_g.at[i]))
        pastes = []
        for k in range(2):
            pastes.append(
                pltpu.async_copy(num_hbm.at[k, pl.ds(base, _BPW)],
                                 num_v.at[k], sem_p.at[k]))
        writes = []
        for i, g in enumerate(gathers):
            g.wait()
            writes.append(
                pltpu.async_copy(rows_v.at[i],
                                 out_hbm.at[i, pl.ds(base, _BPW)], sem_w.at[i]))
        for k, p in enumerate(pastes):
            p.wait()
            writes.append(
                pltpu.async_copy(num_v.at[k],
                                 out_hbm.at[4 + k, pl.ds(base, _BPW)],
                                 sem_w.at[4 + k]))
        for w in writes:
            w.wait()

    return body(i0, i1, i2, i3, num, t0, t1, t2, t3)


def _num_body(x0_ref, x1_ref, g0_ref, be0_ref, w0_ref, b0_ref,
              g1_ref, be1_ref, w1_ref, b1_ref, out_ref):
    j = pl.program_id(0)

    def compute(x, g, be, w, b):
        mean = jnp.mean(x, axis=0, keepdims=True)
        xc = x - mean
        var = jnp.mean(xc * xc, axis=0, keepdims=True)
        h = xc * (g * lax.rsqrt(var + 1e-5)) + be
        out_ref[0] = jnp.dot(h, w, preferred_element_type=jnp.float32) + b

    @pl.when(j == 0)
    def _():
        compute(x0_ref[...], g0_ref[...], be0_ref[...], w0_ref[...], b0_ref[...])

    @pl.when(j == 1)
    def _():
        compute(x1_ref[...], g1_ref[...], be1_ref[...], w1_ref[...], b1_ref[...])


def kernel(x_cat0, x_cat1, x_cat2, x_cat3, x_num0, x_num1,
           table0, table1, table2, table3,
           gamma0, beta0, W0, b0, gamma1, beta1, W1, b1):
    full = pl.BlockSpec(memory_space=pltpu.MemorySpace.VMEM)
    num = pl.pallas_call(
        _num_body,
        grid=(2,),
        in_specs=[full] * 10,
        out_specs=pl.BlockSpec((1, BATCH, D_MODEL), lambda j: (j, 0, 0)),
        out_shape=jax.ShapeDtypeStruct((2, BATCH, D_MODEL), jnp.float32),
    )(x_num0, x_num1,
      gamma0.reshape(1, NUM_DIM), beta0.reshape(1, NUM_DIM), W0,
      b0.reshape(1, D_MODEL),
      gamma1.reshape(1, NUM_DIM), beta1.reshape(1, NUM_DIM), W1,
      b1.reshape(1, D_MODEL))

    return _assemble_sc(x_cat0.astype(jnp.int32), x_cat1.astype(jnp.int32),
                        x_cat2.astype(jnp.int32), x_cat3.astype(jnp.int32),
                        num, table0, table1, table2, table3)


# speedup vs baseline: 4.8549x; 1.2199x over previous
import functools

import jax
import jax.numpy as jnp
from jax import lax
from jax.experimental import pallas as pl
from jax.experimental.pallas import tpu as pltpu
from jax.experimental.pallas import tpu_sc as plsc

D_MODEL = 128
BATCH = 4096
NUM_DIM = 512

_NC = 2
_NS = 16
_NW = _NC * _NS
_BPW = BATCH // _NW


def _gather_sc(i0, i1, i2, i3, t0, t1, t2, t3):
    mesh = plsc.VectorSubcoreMesh(core_axis_name="c", subcore_axis_name="s")

    @functools.partial(
        pl.kernel,
        mesh=mesh,
        out_type=jax.ShapeDtypeStruct((6, BATCH, D_MODEL), jnp.float32),
        scratch_types=[
            pltpu.VMEM((4, _BPW), jnp.int32),
            pltpu.VMEM((4, _BPW, D_MODEL), jnp.float32),
            pltpu.SemaphoreType.DMA((4,)),
            pltpu.SemaphoreType.DMA((4,)),
        ],
    )
    def body(ih0, ih1, ih2, ih3, tb0, tb1, tb2, tb3, out_hbm,
             idx_v, rows_v, sem_g, sem_w):
        wid = lax.axis_index("s") * _NC + lax.axis_index("c")
        base = wid * _BPW
        for i, ih in enumerate((ih0, ih1, ih2, ih3)):
            pltpu.sync_copy(ih.at[pl.ds(base, _BPW)], idx_v.at[i])
        gathers = []
        for i, tbl in enumerate((tb0, tb1, tb2, tb3)):
            gathers.append(
                pltpu.async_copy(tbl.at[idx_v.at[i]], rows_v.at[i], sem_g.at[i]))
        writes = []
        for i, g in enumerate(gathers):
            g.wait()
            writes.append(
                pltpu.async_copy(rows_v.at[i],
                                 out_hbm.at[i, pl.ds(base, _BPW)], sem_w.at[i]))
        for w in writes:
            w.wait()

    return body(i0, i1, i2, i3, t0, t1, t2, t3)


def _num_body(x0_ref, x1_ref, g0_ref, be0_ref, w0_ref, b0_ref,
              g1_ref, be1_ref, w1_ref, b1_ref, out_ref):
    j = pl.program_id(0)

    def compute(x, g, be, w, b):
        mean = jnp.mean(x, axis=0, keepdims=True)
        xc = x - mean
        var = jnp.mean(xc * xc, axis=0, keepdims=True)
        h = xc * (g * lax.rsqrt(var + 1e-5)) + be
        out_ref[0] = jnp.dot(h, w, preferred_element_type=jnp.float32) + b

    @pl.when(j == 0)
    def _():
        compute(x0_ref[...], g0_ref[...], be0_ref[...], w0_ref[...], b0_ref[...])

    @pl.when(j == 1)
    def _():
        compute(x1_ref[...], g1_ref[...], be1_ref[...], w1_ref[...], b1_ref[...])


def _paste_body(buf_ref, num_ref, out_ref):
    out_ref[...] = num_ref[...]


def kernel(x_cat0, x_cat1, x_cat2, x_cat3, x_num0, x_num1,
           table0, table1, table2, table3,
           gamma0, beta0, W0, b0, gamma1, beta1, W1, b1):
    buf = _gather_sc(x_cat0.astype(jnp.int32), x_cat1.astype(jnp.int32),
                     x_cat2.astype(jnp.int32), x_cat3.astype(jnp.int32),
                     table0, table1, table2, table3)

    full = pl.BlockSpec(memory_space=pltpu.MemorySpace.VMEM)
    num = pl.pallas_call(
        _num_body,
        grid=(2,),
        in_specs=[full] * 10,
        out_specs=pl.BlockSpec((1, BATCH, D_MODEL), lambda j: (j, 0, 0)),
        out_shape=jax.ShapeDtypeStruct((2, BATCH, D_MODEL), jnp.float32),
    )(x_num0, x_num1,
      gamma0.reshape(1, NUM_DIM), beta0.reshape(1, NUM_DIM), W0,
      b0.reshape(1, D_MODEL),
      gamma1.reshape(1, NUM_DIM), beta1.reshape(1, NUM_DIM), W1,
      b1.reshape(1, D_MODEL))

    return pl.pallas_call(
        _paste_body,
        grid=(2,),
        in_specs=[
            pl.BlockSpec(memory_space=pltpu.MemorySpace.HBM),
            pl.BlockSpec((1, BATCH, D_MODEL), lambda j: (j, 0, 0)),
        ],
        out_specs=pl.BlockSpec((1, BATCH, D_MODEL), lambda j: (4 + j, 0, 0)),
        out_shape=jax.ShapeDtypeStruct((6, BATCH, D_MODEL), jnp.float32),
        input_output_aliases={0: 0},
    )(buf, num)
